# baseline (speedup 1.0000x reference)
import functools

import jax
import jax.numpy as jnp
from jax import lax
from jax.experimental import pallas as pl
from jax.experimental.pallas import tpu as pltpu
from jax.experimental.pallas import tpu_sc as plsc

_CB = 128
_E = 32
_L = 8
_BATCH = 16384


_BLK_B = 2048


def _quant_body(cb_ref, lat_ref, q_ref):
    cb = cb_ref[...]
    cnorm = jnp.sum(cb * cb, axis=1, keepdims=True)
    for l in range(_L):
        z = lat_ref[l]
        d = -2.0 * lax.dot_general(
            cb, z, (((1,), (0,)), ((), ())),
            preferred_element_type=jnp.float32)
        d = d + cnorm
        m = jnp.min(d, axis=0, keepdims=True)
        ii = lax.broadcasted_iota(jnp.int32, d.shape, 0)
        idx = jnp.min(jnp.where(d == m, ii, _CB), axis=0, keepdims=True)
        e = (ii == idx).astype(jnp.float32)
        q_ref[l] = lax.dot_general(
            cb, e, (((0,), (0,)), ((), ())),
            preferred_element_type=jnp.float32)


def _tc_quantize(latent_t, codebook):
    blk = pl.BlockSpec((_L, _E, _BLK_B), lambda i: (0, 0, i))
    return pl.pallas_call(
        _quant_body,
        grid=(_BATCH // _BLK_B,),
        in_specs=[pl.BlockSpec((_CB, _E), lambda i: (0, 0)), blk],
        out_specs=blk,
        out_shape=jax.ShapeDtypeStruct((_L, _E, _BATCH), jnp.float32),
    )(codebook, latent_t)


_NC, _NS = 2, 16
_NW = _NC * _NS
_CHK = _BATCH // 8


def _sc_dup_body(q_hbm, out, buf, sem):
    wid = lax.axis_index("s") * _NC + lax.axis_index("c")
    l = wid // 4
    r0 = (wid % 4) * 8
    for j in range(2):
        src = q_hbm.at[l, pl.ds(r0 + j * 4, 4)]
        dst = out.at[l, pl.ds(r0 + j * 4, 4)]
        pltpu.async_copy(src, buf, sem).wait()
        pltpu.async_copy(buf, dst, sem).wait()


@functools.cache
def _sc_dup_kernel():
    return pl.kernel(
        _sc_dup_body,
        out_type=jax.ShapeDtypeStruct((_L, _E, _BATCH), jnp.float32),
        mesh=plsc.VectorSubcoreMesh(core_axis_name="c", subcore_axis_name="s"),
        scratch_types=[
            pltpu.VMEM((4, _BATCH), jnp.float32),
            pltpu.SemaphoreType.DMA,
        ],
    )


_BLK_W = 512


def _bcast_body(cbt_ref, out_ref):
    out_ref[...] = jnp.broadcast_to(cbt_ref[...][None], (_BLK_W, _E, _CB))


def _tc_broadcast(cbt):
    return pl.pallas_call(
        _bcast_body,
        grid=(_BATCH // _BLK_W,),
        in_specs=[pl.BlockSpec((_E, _CB), lambda i: (0, 0))],
        out_specs=pl.BlockSpec((_BLK_W, _E, _CB), lambda i: (i, 0, 0)),
        out_shape=jax.ShapeDtypeStruct((_BATCH, _E, _CB), jnp.float32),
    )(cbt)


def kernel(latent, codebook):
    latent_t = jnp.transpose(latent, (1, 2, 0))
    q = _tc_quantize(latent_t, codebook)
    p = _sc_dup_kernel()(q)
    cbw = _tc_broadcast(codebook.T)
    policy = jnp.transpose(p, (2, 0, 1))
    quantized = jnp.transpose(q, (2, 0, 1))
    codebook_weight = jnp.swapaxes(cbw, 1, 2)
    return policy, quantized, codebook_weight
